# Initial kernel scaffold; baseline (speedup 1.0000x reference)
import jax
import jax.numpy as jnp
from jax.experimental import pallas as pl


def kernel(features, edge_index, W1, b1, W2, b2, W3, b3, W4, b4, Wp, bp):
    raise NotImplementedError("write your pallas kernel here")


# speedup vs baseline: 2.8761x; 2.8761x over previous
import functools

import jax
import jax.numpy as jnp
from jax import lax
from jax.experimental import pallas as pl
from jax.experimental.pallas import tpu as pltpu
from jax.experimental.pallas import tpu_sc as plsc

N = 10000
D = 256
E = 160000
NPAD = 10240
EPAD = 163840
TS = 16
CH = 80
K = 128
RPT = NPAD // TS
R = 1280
GRID = NPAD // R

f32 = jnp.float32


def _sc_mesh():
  return plsc.VectorSubcoreMesh(
      core_axis_name="c", subcore_axis_name="s", num_cores=2, num_subcores=TS)


def _hist_body(edges_hbm, ones_hbm, zeros_hbm, deg_hbm, idx_v, ones_v, acc):
  c = lax.axis_index("c")
  s = lax.axis_index("s")
  base = s * RPT
  pltpu.sync_copy(zeros_hbm, acc.at[pl.ds(base, RPT)])
  pltpu.sync_copy(ones_hbm, ones_v)
  pltpu.sync_copy(edges_hbm.at[c].at[s], idx_v)
  plsc.subcore_barrier()

  def body(j, carry):
    pltpu.sync_copy(ones_v, acc.at[idx_v.at[j]], add=True)
    return carry

  lax.fori_loop(0, CH, body, 0)
  plsc.subcore_barrier()
  pltpu.sync_copy(acc.at[pl.ds(base, RPT)],
                  deg_hbm.at[c].at[pl.ds(base, RPT)])


@functools.lru_cache(maxsize=None)
def _hist_kernel():
  return pl.kernel(
      _hist_body,
      out_type=jax.ShapeDtypeStruct((2, NPAD, 128), f32),
      mesh=_sc_mesh(),
      scratch_types=[
          pltpu.VMEM((CH, K), jnp.int32),
          pltpu.VMEM((K, 128), f32),
          pltpu.VMEM_SHARED((NPAD, 128), f32),
      ],
  )


def _hist(*args):
  return _hist_kernel()(*args)


def _spmm_body(x_hbm, src_hbm, dst_hbm, zeros_hbm, out_hbm,
               srcv, dstv, rows, acc, sem):
  c = lax.axis_index("c")
  s = lax.axis_index("s")
  base = s * RPT
  pltpu.sync_copy(src_hbm.at[s], srcv)
  pltpu.sync_copy(dst_hbm.at[s], dstv)
  pltpu.sync_copy(zeros_hbm, acc.at[pl.ds(base, RPT)])
  plsc.subcore_barrier()

  xc = x_hbm.at[c]

  def body(j, carry):
    pltpu.async_copy(xc.at[srcv.at[j]], rows, sem).wait()
    pltpu.sync_copy(rows, acc.at[dstv.at[j]], add=True)
    return carry

  lax.fori_loop(0, CH, body, 0)
  plsc.subcore_barrier()
  pltpu.sync_copy(acc.at[pl.ds(base, RPT)],
                  out_hbm.at[c].at[pl.ds(base, RPT)])


@functools.lru_cache(maxsize=None)
def _spmm_kernel():
  return pl.kernel(
      _spmm_body,
      out_type=jax.ShapeDtypeStruct((2, NPAD, 128), f32),
      mesh=_sc_mesh(),
      scratch_types=[
          pltpu.VMEM((CH, K), jnp.int32),
          pltpu.VMEM((CH, K), jnp.int32),
          pltpu.VMEM((K, 128), f32),
          pltpu.VMEM_SHARED((NPAD, 128), f32),
          pltpu.SemaphoreType.DMA,
      ],
  )


def _spmm(*args):
  return _spmm_kernel()(*args)


def _prep_body(dego_ref, degi_ref, f_ref, ns_ref, nd_ref, x_ref):
  ns = lax.rsqrt(jnp.maximum(dego_ref[:, :1], 1.0))
  nd = lax.rsqrt(jnp.maximum(degi_ref[:, :1], 1.0))
  ns_ref[...] = jnp.broadcast_to(ns, (R, 16))
  nd_ref[...] = jnp.broadcast_to(nd, (R, 16))
  x_ref[0] = f_ref[0] * ns
  x_ref[1] = f_ref[1] * ns


def _prep(deg_o, deg_i, feat_s):
  return pl.pallas_call(
      _prep_body,
      grid=(GRID,),
      in_specs=[
          pl.BlockSpec((R, 128), lambda i: (i, 0)),
          pl.BlockSpec((R, 128), lambda i: (i, 0)),
          pl.BlockSpec((2, R, 128), lambda i: (0, i, 0)),
      ],
      out_specs=[
          pl.BlockSpec((R, 16), lambda i: (i, 0)),
          pl.BlockSpec((R, 16), lambda i: (i, 0)),
          pl.BlockSpec((2, R, 128), lambda i: (0, i, 0)),
      ],
      out_shape=[
          jax.ShapeDtypeStruct((NPAD, 16), f32),
          jax.ShapeDtypeStruct((NPAD, 16), f32),
          jax.ShapeDtypeStruct((2, NPAD, 128), f32),
      ],
  )(deg_o, deg_i, feat_s)


def _layer_body(x_ref, w_ref, b_ref, nd_ref, ns_ref, o_ref):
  z = jnp.dot(x_ref[0], w_ref[0], preferred_element_type=f32)
  z = z + jnp.dot(x_ref[1], w_ref[1], preferred_element_type=f32)
  h = jnp.maximum(z * nd_ref[:, :1] + b_ref[...], 0.0)
  hs = h * ns_ref[:, :1]
  o_ref[0] = hs[:, :128]
  o_ref[1] = hs[:, 128:]


def _layer(agg, w2, b2, nd, ns):
  return pl.pallas_call(
      _layer_body,
      grid=(GRID,),
      in_specs=[
          pl.BlockSpec((2, R, 128), lambda i: (0, i, 0)),
          pl.BlockSpec((2, 128, D), lambda i: (0, 0, 0)),
          pl.BlockSpec((1, D), lambda i: (0, 0)),
          pl.BlockSpec((R, 16), lambda i: (i, 0)),
          pl.BlockSpec((R, 16), lambda i: (i, 0)),
      ],
      out_specs=pl.BlockSpec((2, R, 128), lambda i: (0, i, 0)),
      out_shape=jax.ShapeDtypeStruct((2, NPAD, 128), f32),
  )(agg, w2, b2, nd, ns)


def _pool_body(x_ref, wp_ref, bp_ref, o_ref):
  s0 = jnp.sum(x_ref[0, :N, :], axis=0, keepdims=True) * (1.0 / N)
  s1 = jnp.sum(x_ref[1, :N, :], axis=0, keepdims=True) * (1.0 / N)
  o_ref[...] = (jnp.dot(s0, wp_ref[0], preferred_element_type=f32)
                + jnp.dot(s1, wp_ref[1], preferred_element_type=f32)
                + bp_ref[...])


def _pool(x, wp2, bp2):
  return pl.pallas_call(
      _pool_body,
      out_shape=jax.ShapeDtypeStruct((1, 1), f32),
  )(x, wp2, bp2)


def kernel(features, edge_index, W1, b1, W2, b2, W3, b3, W4, b4, Wp, bp):
  src = edge_index[0].astype(jnp.int32)
  dst = edge_index[1].astype(jnp.int32)
  fill = jnp.full((EPAD - E,), N, jnp.int32)
  src_p = jnp.concatenate([src, fill]).reshape(TS, CH, K)
  dst_p = jnp.concatenate([dst, fill]).reshape(TS, CH, K)

  ones128 = jnp.ones((K, 128), f32)
  zeros128 = jnp.zeros((RPT, 128), f32)

  edges_p = jnp.stack([src_p, dst_p])
  deg = _hist(edges_p, ones128, zeros128)
  deg_o, deg_i = deg[0], deg[1]

  featp = jnp.pad(features, ((0, NPAD - N), (0, 0)))
  feat_s = featp.reshape(NPAD, 2, 128).transpose(1, 0, 2)
  ns, nd, x = _prep(deg_o, deg_i, feat_s)

  ones_col = jnp.ones((NPAD, 16), f32)
  for W, b, nsv in ((W1, b1, ns), (W2, b2, ns), (W3, b3, ns),
                    (W4, b4, ones_col)):
    agg = _spmm(x, src_p, dst_p, zeros128)
    x = _layer(agg, W.reshape(2, 128, D), b.reshape(1, D), nd, nsv)

  return _pool(x, Wp.reshape(2, 128, 1), bp.reshape(1, 1))


# speedup vs baseline: 3.1578x; 1.0980x over previous
import functools

import jax
import jax.numpy as jnp
from jax import lax
from jax.experimental import pallas as pl
from jax.experimental.pallas import tpu as pltpu
from jax.experimental.pallas import tpu_sc as plsc

N = 10000
D = 256
E = 160000
NPAD = 10240
EPAD = 163840
TS = 16
CH = 80
K = 128
RPT = NPAD // TS
R = 1280
GRID = NPAD // R

f32 = jnp.float32


def _sc_mesh():
  return plsc.VectorSubcoreMesh(
      core_axis_name="c", subcore_axis_name="s", num_cores=2, num_subcores=TS)


def _hist_body(edges_hbm, ones_hbm, zeros_hbm, deg_hbm, idx_v, ones_v, acc):
  c = lax.axis_index("c")
  s = lax.axis_index("s")
  base = s * RPT
  pltpu.sync_copy(zeros_hbm, acc.at[pl.ds(base, RPT)])
  pltpu.sync_copy(ones_hbm, ones_v)
  pltpu.sync_copy(edges_hbm.at[c].at[s], idx_v)
  plsc.subcore_barrier()

  def body(j, carry):
    pltpu.sync_copy(ones_v, acc.at[idx_v.at[j]], add=True)
    return carry

  lax.fori_loop(0, CH, body, 0)
  plsc.subcore_barrier()
  pltpu.sync_copy(acc.at[pl.ds(base, RPT)],
                  deg_hbm.at[c].at[pl.ds(base, RPT)])


@functools.lru_cache(maxsize=None)
def _hist_kernel():
  return pl.kernel(
      _hist_body,
      out_type=jax.ShapeDtypeStruct((2, NPAD, 128), f32),
      mesh=_sc_mesh(),
      scratch_types=[
          pltpu.VMEM((CH, K), jnp.int32),
          pltpu.VMEM((K, 128), f32),
          pltpu.VMEM_SHARED((NPAD, 128), f32),
      ],
  )


def _hist(*args):
  return _hist_kernel()(*args)


def _spmm_body(x_hbm, src_hbm, dst_hbm, zeros_hbm, out_hbm,
               srcv, dstw, rows, acc,
               semg0, semg1, sems0, sems1, semi0, semi1):
  c = lax.axis_index("c")
  s = lax.axis_index("s")
  base = s * RPT
  pltpu.sync_copy(src_hbm.at[s], srcv)
  pltpu.sync_copy(zeros_hbm, acc.at[pl.ds(base, RPT)])
  dsts = dst_hbm.at[s]
  xc = x_hbm.at[c]
  semg = (semg0, semg1)
  sems = (sems0, sems1)
  semi = (semi0, semi1)

  def fire_i(j, slot):
    pltpu.async_copy(dsts.at[j], dstw.at[slot], semi[slot % 2])

  def drain_i(j, slot):
    pltpu.make_async_copy(dsts.at[j], dstw.at[slot], semi[slot % 2]).wait()

  def fire_g(j, p):
    pltpu.async_copy(xc.at[srcv.at[j]], rows.at[p], semg[p])

  def drain_g(j, p):
    pltpu.make_async_copy(xc.at[srcv.at[j]], rows.at[p], semg[p]).wait()

  def fire_s(p, slot):
    pltpu.async_copy(rows.at[p], acc.at[dstw.at[slot]], sems[p], add=True)

  def drain_s(p, slot):
    pltpu.make_async_copy(rows.at[p], acc.at[dstw.at[slot]], sems[p]).wait()

  fire_i(0, 0)
  fire_i(1, 1)
  fire_g(0, 0)
  plsc.subcore_barrier()

  def step(j, b, first=False, fi=True, fg=True):
    p = b % 2
    d = b % 4
    drain_g(j, p)
    drain_i(j, d)
    fire_s(p, d)
    if not first:
      drain_s(1 - p, (b - 1) % 4)
    if fi:
      fire_i(j + 2, (b + 2) % 4)
    if fg:
      fire_g(j + 1, 1 - p)

  step(0, 0, first=True)
  step(1, 1)
  step(2, 2)
  step(3, 3)

  def body(t, carry):
    j0 = 4 * t
    step(j0 + 0, 0)
    step(j0 + 1, 1)
    step(j0 + 2, 2)
    step(j0 + 3, 3)
    return carry

  lax.fori_loop(1, CH // 4 - 1, body, 0)

  step(CH - 4, 0)
  step(CH - 3, 1)
  step(CH - 2, 2, fi=False)
  step(CH - 1, 3, fi=False, fg=False)
  drain_s(1, 3)
  plsc.subcore_barrier()
  pltpu.sync_copy(acc.at[pl.ds(base, RPT)],
                  out_hbm.at[c].at[pl.ds(base, RPT)])


@functools.lru_cache(maxsize=None)
def _spmm_kernel():
  return pl.kernel(
      _spmm_body,
      out_type=jax.ShapeDtypeStruct((2, NPAD, 128), f32),
      mesh=_sc_mesh(),
      scratch_types=[
          pltpu.VMEM((CH, K), jnp.int32),
          pltpu.VMEM((4, K), jnp.int32),
          pltpu.VMEM((2, K, 128), f32),
          pltpu.VMEM_SHARED((NPAD, 128), f32),
          pltpu.SemaphoreType.DMA,
          pltpu.SemaphoreType.DMA,
          pltpu.SemaphoreType.DMA,
          pltpu.SemaphoreType.DMA,
          pltpu.SemaphoreType.DMA,
          pltpu.SemaphoreType.DMA,
      ],
  )


def _spmm(*args):
  return _spmm_kernel()(*args)


def _prep_body(dego_ref, degi_ref, f_ref, ns_ref, nd_ref, x_ref):
  ns = lax.rsqrt(jnp.maximum(dego_ref[:, :1], 1.0))
  nd = lax.rsqrt(jnp.maximum(degi_ref[:, :1], 1.0))
  ns_ref[...] = jnp.broadcast_to(ns, (R, 16))
  nd_ref[...] = jnp.broadcast_to(nd, (R, 16))
  x_ref[0] = f_ref[0] * ns
  x_ref[1] = f_ref[1] * ns


def _prep(deg_o, deg_i, feat_s):
  return pl.pallas_call(
      _prep_body,
      grid=(GRID,),
      in_specs=[
          pl.BlockSpec((R, 128), lambda i: (i, 0)),
          pl.BlockSpec((R, 128), lambda i: (i, 0)),
          pl.BlockSpec((2, R, 128), lambda i: (0, i, 0)),
      ],
      out_specs=[
          pl.BlockSpec((R, 16), lambda i: (i, 0)),
          pl.BlockSpec((R, 16), lambda i: (i, 0)),
          pl.BlockSpec((2, R, 128), lambda i: (0, i, 0)),
      ],
      out_shape=[
          jax.ShapeDtypeStruct((NPAD, 16), f32),
          jax.ShapeDtypeStruct((NPAD, 16), f32),
          jax.ShapeDtypeStruct((2, NPAD, 128), f32),
      ],
  )(deg_o, deg_i, feat_s)


def _layer_body(x_ref, w_ref, b_ref, nd_ref, ns_ref, o_ref):
  hp = lax.Precision.HIGHEST
  z = jnp.dot(x_ref[0], w_ref[0], preferred_element_type=f32, precision=hp)
  z = z + jnp.dot(x_ref[1], w_ref[1], preferred_element_type=f32, precision=hp)
  h = jnp.maximum(z * nd_ref[:, :1] + b_ref[...], 0.0)
  hs = h * ns_ref[:, :1]
  o_ref[0] = hs[:, :128]
  o_ref[1] = hs[:, 128:]


def _layer(agg, w2, b2, nd, ns):
  return pl.pallas_call(
      _layer_body,
      grid=(GRID,),
      in_specs=[
          pl.BlockSpec((2, R, 128), lambda i: (0, i, 0)),
          pl.BlockSpec((2, 128, D), lambda i: (0, 0, 0)),
          pl.BlockSpec((1, D), lambda i: (0, 0)),
          pl.BlockSpec((R, 16), lambda i: (i, 0)),
          pl.BlockSpec((R, 16), lambda i: (i, 0)),
      ],
      out_specs=pl.BlockSpec((2, R, 128), lambda i: (0, i, 0)),
      out_shape=jax.ShapeDtypeStruct((2, NPAD, 128), f32),
  )(agg, w2, b2, nd, ns)


def _pool_body(x_ref, wp_ref, bp_ref, o_ref):
  s0 = jnp.sum(x_ref[0, :N, :], axis=0, keepdims=True) * (1.0 / N)
  s1 = jnp.sum(x_ref[1, :N, :], axis=0, keepdims=True) * (1.0 / N)
  hp = lax.Precision.DEFAULT
  o_ref[...] = (jnp.dot(s0, wp_ref[0], preferred_element_type=f32, precision=hp)
                + jnp.dot(s1, wp_ref[1], preferred_element_type=f32, precision=hp)
                + bp_ref[...])


def _pool(x, wp2, bp2):
  return pl.pallas_call(
      _pool_body,
      out_shape=jax.ShapeDtypeStruct((1, 1), f32),
  )(x, wp2, bp2)


def kernel(features, edge_index, W1, b1, W2, b2, W3, b3, W4, b4, Wp, bp):
  src = edge_index[0].astype(jnp.int32)
  dst = edge_index[1].astype(jnp.int32)
  fill = jnp.full((EPAD - E,), N, jnp.int32)
  src_p = jnp.concatenate([src, fill]).reshape(TS, CH, K)
  dst_p = jnp.concatenate([dst, fill]).reshape(TS, CH, K)

  ones128 = jnp.ones((K, 128), f32)
  zeros128 = jnp.zeros((RPT, 128), f32)

  edges_p = jnp.stack([src_p, dst_p])
  deg = _hist(edges_p, ones128, zeros128)
  deg_o, deg_i = deg[0], deg[1]

  featp = jnp.pad(features, ((0, NPAD - N), (0, 0)))
  feat_s = featp.reshape(NPAD, 2, 128).transpose(1, 0, 2)
  ns, nd, x = _prep(deg_o, deg_i, feat_s)

  ones_col = jnp.ones((NPAD, 16), f32)
  for W, b, nsv in ((W1, b1, ns), (W2, b2, ns), (W3, b3, ns),
                    (W4, b4, ones_col)):
    agg = _spmm(x, src_p, dst_p, zeros128)
    x = _layer(agg, W.reshape(2, 128, D), b.reshape(1, D), nd, nsv)

  return _pool(x, Wp.reshape(2, 128, 1), bp.reshape(1, 1))
